# baseline (speedup 1.0000x reference)
import jax
import jax.numpy as jnp
from jax import lax
from jax.experimental import pallas as pl
from jax.experimental.pallas import tpu as pltpu
from jax.experimental.pallas import tpu_sc as plsc

VOCAB = 1000000
DIM = 64
CTX = 8192

_NC = 2
_NS = 16
_NW = _NC * _NS
_BPW = CTX // _NW
_CHUNK = 128
_NCHUNK = _BPW // _CHUNK


def _body(x_hbm, word_hbm, pos_hbm, out_hbm, idx_v, rows_v, sem):
    wid = lax.axis_index("s") * _NC + lax.axis_index("c")
    base = wid * _BPW
    pltpu.sync_copy(x_hbm.at[pl.ds(wid * _NCHUNK, _NCHUNK)], idx_v)
    pltpu.sync_copy(pos_hbm.at[pl.ds(base, _BPW)], rows_v)
    copies = [
        pltpu.async_copy(
            word_hbm.at[idx_v.at[j]],
            rows_v.at[pl.ds(j * _CHUNK, _CHUNK)],
            sem,
            add=True,
        )
        for j in range(_NCHUNK)
    ]
    for c in copies:
        c.wait()
    pltpu.sync_copy(rows_v, out_hbm.at[pl.ds(base, _BPW)])


@jax.jit
def _embed(x2d, word_table, pos_table):
    mesh = plsc.VectorSubcoreMesh(core_axis_name="c", subcore_axis_name="s")
    return pl.kernel(
        _body,
        out_type=jax.ShapeDtypeStruct((CTX, DIM), jnp.float32),
        mesh=mesh,
        scratch_types=[
            pltpu.VMEM((_NCHUNK, _CHUNK), jnp.int32),
            pltpu.VMEM((_BPW, DIM), jnp.float32),
            pltpu.SemaphoreType.DMA,
        ],
        compiler_params=pltpu.CompilerParams(use_tc_tiling_on_sc=False),
    )(x2d, word_table, pos_table)


def kernel(x, word_table, pos_table):
    x2d = x.reshape(CTX // _CHUNK, _CHUNK)
    return _embed(x2d, word_table, pos_table)


# speedup vs baseline: 1.6950x; 1.6950x over previous
import jax
import jax.numpy as jnp
from jax import lax
from jax.experimental import pallas as pl
from jax.experimental.pallas import tpu as pltpu
from jax.experimental.pallas import tpu_sc as plsc

VOCAB = 1000000
DIM = 64
CTX = 8192

_NC = 2
_NS = 16
_NW = _NC * _NS
_BPW = CTX // _NW
_LANES = 16


def _body(x_hbm, word_hbm, pos_hbm, out_hbm, idx_s, idx_v, rows_v, pos_v, sem):
    wid = lax.axis_index("s") * _NC + lax.axis_index("c")
    base = wid * _BPW
    pltpu.sync_copy(x_hbm.at[pl.ds(base, _BPW)], idx_v)
    pltpu.sync_copy(pos_hbm.at[pl.ds(base, _BPW)], pos_v)

    def fire(g, _):
        v = idx_v[pl.ds(g * _LANES, _LANES)]
        for k in range(_LANES):
            t = v[k]
            pltpu.async_copy(
                word_hbm.at[pl.ds(t, 1)],
                rows_v.at[pl.ds(g * _LANES + k, 1)],
                sem,
            )
        return 0

    lax.fori_loop(0, _BPW // _LANES, fire, 0)
    pltpu.make_async_copy(word_hbm.at[pl.ds(0, _BPW)], rows_v, sem).wait()

    def add_row(i, _):
        for j in range(DIM // _LANES):
            sl = pl.ds(j * _LANES, _LANES)
            rows_v[i, sl] = rows_v[i, sl] + pos_v[i, sl]
        return 0

    lax.fori_loop(0, _BPW, add_row, 0)
    pltpu.sync_copy(rows_v, out_hbm.at[pl.ds(base, _BPW)])


@jax.jit
def _embed(x, word_table, pos_table):
    mesh = plsc.VectorSubcoreMesh(core_axis_name="c", subcore_axis_name="s")
    return pl.kernel(
        _body,
        out_type=jax.ShapeDtypeStruct((CTX, DIM), jnp.float32),
        mesh=mesh,
        scratch_types=[
            pltpu.SMEM((_BPW,), jnp.int32),
            pltpu.VMEM((_BPW,), jnp.int32),
            pltpu.VMEM((_BPW, DIM), jnp.float32),
            pltpu.VMEM((_BPW, DIM), jnp.float32),
            pltpu.SemaphoreType.DMA,
        ],
    )(x, word_table, pos_table)


def kernel(x, word_table, pos_table):
    return _embed(x, word_table, pos_table)


# speedup vs baseline: 3.8224x; 2.2551x over previous
import jax
import jax.numpy as jnp
from jax import lax
from jax.experimental import pallas as pl
from jax.experimental.pallas import tpu as pltpu
from jax.experimental.pallas import tpu_sc as plsc

VOCAB = 1000000
DIM = 64
CTX = 8192

_NC = 2
_NS = 16
_NW = _NC * _NS
_BPW = CTX // _NW
_LANES = 16
_RING = 4
_DCH = DIM // _LANES


def _fire(wt_hbm, grp_v, sem, b, t):
    off = (t >> 7) * 128
    pltpu.async_copy(
        wt_hbm.at[:, pl.ds(off, 128)],
        grp_v.at[pl.ds(b * DIM, DIM)],
        sem,
    )


def _body(x_hbm, wt_hbm, pos_hbm, out_hbm, idx_v, grp_v, pos_v, rows_v, sem):
    wid = lax.axis_index("s") * _NC + lax.axis_index("c")
    base = wid * _BPW
    pltpu.sync_copy(x_hbm.at[pl.ds(base, _BPW)], idx_v.at[pl.ds(0, _BPW)])
    pltpu.sync_copy(pos_hbm.at[pl.ds(base, _BPW)], pos_v)

    iotas = [
        jax.lax.iota(jnp.int32, _LANES) + g * _LANES for g in range(_DCH)
    ]

    v0 = idx_v[pl.ds(0, _LANES)]
    for b in range(_RING):
        _fire(wt_hbm, grp_v, sem, b, v0[b])

    def round_body(r, _):
        v = idx_v[pl.ds(r * _RING, _LANES)]
        for b in range(_RING):
            s = r * _RING + b
            t = v[b]
            pltpu.make_async_copy(
                wt_hbm.at[:, pl.ds(0, 128)],
                grp_v.at[pl.ds(b * DIM, DIM)],
                sem,
            ).wait()
            vloc = jnp.broadcast_to(t & 127, (_LANES,))
            for g in range(_DCH):
                col = plsc.load_gather(grp_v, [iotas[g] + b * DIM, vloc])
                sl = pl.ds(g * _LANES, _LANES)
                rows_v[s, sl] = col + pos_v[s, sl]
        @pl.when(r < _BPW // _RING - 1)
        def _():
            for b in range(_RING):
                _fire(wt_hbm, grp_v, sem, b, v[_RING + b])
        return 0

    lax.fori_loop(0, _BPW // _RING, round_body, 0)
    pltpu.sync_copy(rows_v, out_hbm.at[pl.ds(base, _BPW)])


@jax.jit
def _embed(x, wt_t, pos_table):
    mesh = plsc.VectorSubcoreMesh(core_axis_name="c", subcore_axis_name="s")
    return pl.kernel(
        _body,
        out_type=jax.ShapeDtypeStruct((CTX, DIM), jnp.float32),
        mesh=mesh,
        scratch_types=[
            pltpu.VMEM((_BPW + _LANES,), jnp.int32),
            pltpu.VMEM((_RING * DIM, 128), jnp.float32),
            pltpu.VMEM((_BPW, DIM), jnp.float32),
            pltpu.VMEM((_BPW, DIM), jnp.float32),
            pltpu.SemaphoreType.DMA,
        ],
        compiler_params=pltpu.CompilerParams(needs_layout_passes=False),
    )(x, wt_t, pos_table)


def kernel(x, word_table, pos_table):
    return _embed(x, word_table.T, pos_table)


# speedup vs baseline: 4.5152x; 1.1813x over previous
import jax
import jax.numpy as jnp
from jax import lax
from jax.experimental import pallas as pl
from jax.experimental.pallas import tpu as pltpu
from jax.experimental.pallas import tpu_sc as plsc

VOCAB = 1000000
DIM = 64
CTX = 8192

_NC = 2
_NS = 16
_NW = _NC * _NS
_BPW = CTX // _NW
_LANES = 16
_RING = 8
_DCH = DIM // _LANES


def _fire(wt_hbm, grp_v, sem, b, t):
    off = (t >> 7) * 128
    pltpu.async_copy(
        wt_hbm.at[:, pl.ds(off, 128)],
        grp_v.at[pl.ds(b * DIM, DIM)],
        sem,
    )


def _body(x_hbm, wt_hbm, pos_hbm, out_hbm, idx_v, grp_v, rows_v, sem):
    wid = lax.axis_index("s") * _NC + lax.axis_index("c")
    base = wid * _BPW
    pltpu.sync_copy(x_hbm.at[pl.ds(base, _BPW)], idx_v.at[pl.ds(0, _BPW)])
    pltpu.sync_copy(pos_hbm.at[pl.ds(base, _BPW)], rows_v)

    iotas = [
        jax.lax.iota(jnp.int32, _LANES) + g * _LANES for g in range(_DCH)
    ]

    v0 = idx_v[pl.ds(0, _LANES)]
    for b in range(_RING):
        _fire(wt_hbm, grp_v, sem, b, v0[b])

    def round_body(r, _):
        v = idx_v[pl.ds(r * _RING, _LANES)]
        for b in range(_RING):
            s = r * _RING + b
            t = v[b]
            pltpu.make_async_copy(
                wt_hbm.at[:, pl.ds(0, 128)],
                grp_v.at[pl.ds(b * DIM, DIM)],
                sem,
            ).wait()
            vloc = jnp.broadcast_to(t & 127, (_LANES,))
            for g in range(_DCH):
                col = plsc.load_gather(grp_v, [iotas[g] + b * DIM, vloc])
                sl = pl.ds(g * _LANES, _LANES)
                rows_v[s, sl] = rows_v[s, sl] + col
        @pl.when(r < _BPW // _RING - 1)
        def _():
            for b in range(_RING):
                _fire(wt_hbm, grp_v, sem, b, v[_RING + b])
        return 0

    lax.fori_loop(0, _BPW // _RING, round_body, 0)
    pltpu.sync_copy(rows_v, out_hbm.at[pl.ds(base, _BPW)])


@jax.jit
def _embed(x, wt_t, pos_table):
    mesh = plsc.VectorSubcoreMesh(core_axis_name="c", subcore_axis_name="s")
    return pl.kernel(
        _body,
        out_type=jax.ShapeDtypeStruct((CTX, DIM), jnp.float32),
        mesh=mesh,
        scratch_types=[
            pltpu.VMEM((_BPW + _LANES,), jnp.int32),
            pltpu.VMEM((_RING * DIM, 128), jnp.float32),
            pltpu.VMEM((_BPW, DIM), jnp.float32),
            pltpu.SemaphoreType.DMA,
        ],
        compiler_params=pltpu.CompilerParams(needs_layout_passes=False),
    )(x, wt_t, pos_table)


def kernel(x, word_table, pos_table):
    return _embed(x, word_table.T, pos_table)
